# baseline (speedup 1.0000x reference)
import jax
import jax.numpy as jnp
from jax.experimental import pallas as pl
from jax.experimental.pallas import tpu as pltpu

BATCH = 16384
TITLE_DIM = 200
NUM_DIM = 36
DOMAIN_DIM = 16
USER_DIM = 24
HIDDEN = 128


def _full(a):
    return pl.BlockSpec(a.shape, lambda i: (0,) * a.ndim)


def _stage1_body(title_ref, num_ref, w1t_ref, w1n_ref, b1_ref, acc_ref):
    bf = jnp.bfloat16
    f32 = jnp.float32
    acc = jnp.dot(w1t_ref[...].astype(bf), title_ref[...].astype(bf),
                  preferred_element_type=f32)
    acc += jnp.dot(w1n_ref[...].astype(bf), num_ref[...].astype(bf),
                   preferred_element_type=f32)
    acc_ref[...] = (acc + b1_ref[...]).astype(bf)


def _stage1(title_t, num_t, w1t_t, w1n_t, b1c, block_m=2048):
    grid = (BATCH // block_m,)
    return pl.pallas_call(
        _stage1_body,
        grid=grid,
        in_specs=[
            pl.BlockSpec((TITLE_DIM, block_m), lambda i: (0, i)),
            pl.BlockSpec((NUM_DIM, block_m), lambda i: (0, i)),
            _full(w1t_t), _full(w1n_t), _full(b1c),
        ],
        out_specs=pl.BlockSpec((HIDDEN, block_m), lambda i: (0, i)),
        out_shape=jax.ShapeDtypeStruct((HIDDEN, BATCH), jnp.bfloat16),
        compiler_params=pltpu.CompilerParams(
            dimension_semantics=("parallel",)),
    )(title_t, num_t, w1t_t, w1n_t, b1c)


def _stage2_body(acc_ref, dom8_ref, mask_ref, usr_ref, w1d_ref, w1u_ref,
                 w2_ref, b2_ref, w3_ref, b3_ref, out_ref):
    bf = jnp.bfloat16
    f32 = jnp.float32
    acc = acc_ref[...].astype(f32)
    gm = dom8_ref[...].astype(bf) * mask_ref[...].astype(bf)
    acc += jax.lax.dot_general(
        w1d_ref[...].astype(bf), gm, (((1,), (1,)), ((), ())),
        preferred_element_type=f32)
    acc += jnp.dot(w1u_ref[...].astype(bf), usr_ref[...].astype(bf),
                   preferred_element_type=f32)
    h1 = jnp.maximum(acc, 0.0)
    h2 = jnp.maximum(
        jnp.dot(w2_ref[...].astype(bf), h1.astype(bf),
                preferred_element_type=f32) + b2_ref[...], 0.0)
    out = jnp.dot(w3_ref[...].astype(bf), h2.astype(bf),
                  preferred_element_type=f32) + b3_ref[...]
    out_ref[...] = out[0]


def _stage2(acc, dom8, mask, usr_t, w1d8_t, w1u_t, w2_t, b2c, w3_t, b3c,
            block_m=2048):
    grid = (BATCH // block_m,)
    return pl.pallas_call(
        _stage2_body,
        grid=grid,
        in_specs=[
            pl.BlockSpec((HIDDEN, block_m), lambda i: (0, i)),
            pl.BlockSpec((block_m, 128), lambda i: (i, 0)),
            pl.BlockSpec((block_m, 128), lambda i: (i, 0)),
            pl.BlockSpec((USER_DIM, block_m), lambda i: (0, i)),
            _full(w1d8_t), _full(w1u_t), _full(w2_t), _full(b2c),
            _full(w3_t), _full(b3c),
        ],
        out_specs=pl.BlockSpec((block_m,), lambda i: (i,)),
        out_shape=jax.ShapeDtypeStruct((BATCH,), jnp.float32),
        compiler_params=pltpu.CompilerParams(
            dimension_semantics=("parallel",)),
    )(acc, dom8, mask, usr_t, w1d8_t, w1u_t, w2_t, b2c, w3_t, b3c)


def kernel(title_emb, numerical_features, domain_ids, user_ids,
           domain_table, user_table, W1, b1, W2, b2, W3, b3):
    dom8_tab = jnp.concatenate([domain_table[k::8, :] for k in range(8)],
                               axis=1)
    ids8 = jax.lax.shift_right_logical(domain_ids, 3)
    mask = (jnp.arange(128, dtype=jnp.int32)[None, :] // DOMAIN_DIM
            == (domain_ids & 7)[:, None]).astype(jnp.bfloat16)
    dom8 = dom8_tab.at[ids8, :].get(mode="promise_in_bounds")
    usr_t = user_table.T.at[:, user_ids].get(mode="promise_in_bounds")
    title_t = title_emb.T
    num_t = numerical_features.T
    w1t_t = W1[:TITLE_DIM].T * 0.5
    w1n_t = W1[TITLE_DIM:TITLE_DIM + NUM_DIM].T
    w1d_t = W1[TITLE_DIM + NUM_DIM:TITLE_DIM + NUM_DIM + DOMAIN_DIM].T
    w1u_t = W1[TITLE_DIM + NUM_DIM + DOMAIN_DIM:].T
    w2_t = W2.T
    w3_t = W3.T
    b1c = b1[:, None]
    b2c = b2[:, None]
    b3c = b3[:, None]
    w1d8_t = jnp.tile(w1d_t, (1, 8))
    acc = _stage1(title_t, num_t, w1t_t, w1n_t, b1c)
    return _stage2(acc, dom8, mask, usr_t, w1d8_t, w1u_t, w2_t, b2c,
                   w3_t, b3c)


# speedup vs baseline: 5.1985x; 5.1985x over previous
import jax
import jax.numpy as jnp
from jax.experimental import pallas as pl
from jax.experimental.pallas import tpu as pltpu

BATCH = 16384
TITLE_DIM = 200
NUM_DIM = 36
DOMAIN_DIM = 16
USER_DIM = 24
HIDDEN = 128


def _full(a):
    return pl.BlockSpec(a.shape, lambda i: (0,) * a.ndim)


def _stage1_body(title_ref, num_ref, w1t_ref, w1n_ref, b1_ref, acc_ref):
    bf = jnp.bfloat16
    f32 = jnp.float32
    acc = jnp.dot(w1t_ref[...].astype(bf), title_ref[...].astype(bf),
                  preferred_element_type=f32)
    acc += jnp.dot(w1n_ref[...].astype(bf), num_ref[...].astype(bf),
                   preferred_element_type=f32)
    acc_ref[...] = (acc + b1_ref[...]).astype(bf)


def _stage1(title_t, num_t, w1t_t, w1n_t, b1c, block_m=2048):
    grid = (BATCH // block_m,)
    return pl.pallas_call(
        _stage1_body,
        grid=grid,
        in_specs=[
            pl.BlockSpec((TITLE_DIM, block_m), lambda i: (0, i)),
            pl.BlockSpec((NUM_DIM, block_m), lambda i: (0, i)),
            _full(w1t_t), _full(w1n_t), _full(b1c),
        ],
        out_specs=pl.BlockSpec((HIDDEN, block_m), lambda i: (0, i)),
        out_shape=jax.ShapeDtypeStruct((HIDDEN, BATCH), jnp.bfloat16),
        compiler_params=pltpu.CompilerParams(
            dimension_semantics=("parallel",)),
    )(title_t, num_t, w1t_t, w1n_t, b1c)


def _stage2_body(acc_ref, dom_ref, usr_ref, w1d_ref, w1u_ref,
                 w2_ref, b2_ref, w3_ref, b3_ref, out_ref):
    bf = jnp.bfloat16
    f32 = jnp.float32
    acc = acc_ref[...].astype(f32)
    acc += jnp.dot(w1d_ref[...].astype(bf), dom_ref[...].astype(bf),
                   preferred_element_type=f32)
    acc += jnp.dot(w1u_ref[...].astype(bf), usr_ref[...].astype(bf),
                   preferred_element_type=f32)
    h1 = jnp.maximum(acc, 0.0)
    h2 = jnp.maximum(
        jnp.dot(w2_ref[...].astype(bf), h1.astype(bf),
                preferred_element_type=f32) + b2_ref[...], 0.0)
    out = jnp.dot(w3_ref[...].astype(bf), h2.astype(bf),
                  preferred_element_type=f32) + b3_ref[...]
    out_ref[...] = out[0]


def _stage2(acc, dom_t, usr_t, w1d_t, w1u_t, w2_t, b2c, w3_t, b3c,
            block_m=2048):
    grid = (BATCH // block_m,)
    return pl.pallas_call(
        _stage2_body,
        grid=grid,
        in_specs=[
            pl.BlockSpec((HIDDEN, block_m), lambda i: (0, i)),
            pl.BlockSpec((DOMAIN_DIM, block_m), lambda i: (0, i)),
            pl.BlockSpec((USER_DIM, block_m), lambda i: (0, i)),
            _full(w1d_t), _full(w1u_t), _full(w2_t), _full(b2c),
            _full(w3_t), _full(b3c),
        ],
        out_specs=pl.BlockSpec((block_m,), lambda i: (i,)),
        out_shape=jax.ShapeDtypeStruct((BATCH,), jnp.float32),
        compiler_params=pltpu.CompilerParams(
            dimension_semantics=("parallel",)),
    )(acc, dom_t, usr_t, w1d_t, w1u_t, w2_t, b2c, w3_t, b3c)


def kernel(title_emb, numerical_features, domain_ids, user_ids,
           domain_table, user_table, W1, b1, W2, b2, W3, b3):
    usr_t = user_table.T.at[:, user_ids].get(mode="promise_in_bounds")
    dom_t = domain_table.T.at[:, domain_ids].get(mode="promise_in_bounds")
    title_t = title_emb.T
    num_t = numerical_features.T
    w1t_t = W1[:TITLE_DIM].T * 0.5
    w1n_t = W1[TITLE_DIM:TITLE_DIM + NUM_DIM].T
    w1d_t = W1[TITLE_DIM + NUM_DIM:TITLE_DIM + NUM_DIM + DOMAIN_DIM].T
    w1u_t = W1[TITLE_DIM + NUM_DIM + DOMAIN_DIM:].T
    w2_t = W2.T
    w3_t = W3.T
    b1c = b1[:, None]
    b2c = b2[:, None]
    b3c = b3[:, None]
    acc = _stage1(title_t, num_t, w1t_t, w1n_t, b1c)
    return _stage2(acc, dom_t, usr_t, w1d_t, w1u_t, w2_t, b2c,
                   w3_t, b3c)


# speedup vs baseline: 5.3548x; 1.0301x over previous
import jax
import jax.numpy as jnp
from jax.experimental import pallas as pl
from jax.experimental.pallas import tpu as pltpu

BATCH = 16384
TITLE_DIM = 200
NUM_DIM = 36
DOMAIN_DIM = 16
USER_DIM = 24
HIDDEN = 128


def _full(a):
    return pl.BlockSpec(a.shape, lambda i: (0,) * a.ndim)


def _stage1_body(title_ref, num_ref, w1t_ref, w1n_ref, b1_ref, acc_ref):
    bf = jnp.bfloat16
    f32 = jnp.float32
    acc = jnp.dot(w1t_ref[...].astype(bf), title_ref[...].astype(bf),
                  preferred_element_type=f32)
    acc += jnp.dot(w1n_ref[...].astype(bf), num_ref[...].astype(bf),
                   preferred_element_type=f32)
    acc_ref[...] = (acc + b1_ref[...]).astype(bf)


def _stage1(title_t, num_t, w1t_t, w1n_t, b1c, block_m=4096):
    grid = (BATCH // block_m,)
    return pl.pallas_call(
        _stage1_body,
        grid=grid,
        in_specs=[
            pl.BlockSpec((TITLE_DIM, block_m), lambda i: (0, i)),
            pl.BlockSpec((NUM_DIM, block_m), lambda i: (0, i)),
            _full(w1t_t), _full(w1n_t), _full(b1c),
        ],
        out_specs=pl.BlockSpec((HIDDEN, block_m), lambda i: (0, i)),
        out_shape=jax.ShapeDtypeStruct((HIDDEN, BATCH), jnp.bfloat16),
        compiler_params=pltpu.CompilerParams(
            dimension_semantics=("parallel",)),
    )(title_t, num_t, w1t_t, w1n_t, b1c)


def _stage2_body(acc_ref, dom_ref, usr_ref, w1d_ref, w1u_ref,
                 w2_ref, b2_ref, w3_ref, b3_ref, out_ref):
    bf = jnp.bfloat16
    f32 = jnp.float32
    acc = acc_ref[...].astype(f32)
    acc += jnp.dot(w1d_ref[...].astype(bf), dom_ref[...].astype(bf),
                   preferred_element_type=f32)
    acc += jnp.dot(w1u_ref[...].astype(bf), usr_ref[...].astype(bf),
                   preferred_element_type=f32)
    h1 = jnp.maximum(acc, 0.0)
    h2 = jnp.maximum(
        jnp.dot(w2_ref[...].astype(bf), h1.astype(bf),
                preferred_element_type=f32) + b2_ref[...], 0.0)
    out = jnp.dot(w3_ref[...].astype(bf), h2.astype(bf),
                  preferred_element_type=f32) + b3_ref[...]
    out_ref[...] = out[0]


def _stage2(acc, dom_t, usr_t, w1d_t, w1u_t, w2_t, b2c, w3_t, b3c,
            block_m=4096):
    grid = (BATCH // block_m,)
    return pl.pallas_call(
        _stage2_body,
        grid=grid,
        in_specs=[
            pl.BlockSpec((HIDDEN, block_m), lambda i: (0, i)),
            pl.BlockSpec((DOMAIN_DIM, block_m), lambda i: (0, i)),
            pl.BlockSpec((USER_DIM, block_m), lambda i: (0, i)),
            _full(w1d_t), _full(w1u_t), _full(w2_t), _full(b2c),
            _full(w3_t), _full(b3c),
        ],
        out_specs=pl.BlockSpec((block_m,), lambda i: (i,)),
        out_shape=jax.ShapeDtypeStruct((BATCH,), jnp.float32),
        compiler_params=pltpu.CompilerParams(
            dimension_semantics=("parallel",)),
    )(acc, dom_t, usr_t, w1d_t, w1u_t, w2_t, b2c, w3_t, b3c)


def kernel(title_emb, numerical_features, domain_ids, user_ids,
           domain_table, user_table, W1, b1, W2, b2, W3, b3):
    usr_t = user_table.T.at[:, user_ids].get(mode="promise_in_bounds")
    dom_t = domain_table.T.at[:, domain_ids].get(mode="promise_in_bounds")
    title_t = title_emb.T
    num_t = numerical_features.T
    w1t_t = W1[:TITLE_DIM].T * 0.5
    w1n_t = W1[TITLE_DIM:TITLE_DIM + NUM_DIM].T
    w1d_t = W1[TITLE_DIM + NUM_DIM:TITLE_DIM + NUM_DIM + DOMAIN_DIM].T
    w1u_t = W1[TITLE_DIM + NUM_DIM + DOMAIN_DIM:].T
    w2_t = W2.T
    w3_t = W3.T
    b1c = b1[:, None]
    b2c = b2[:, None]
    b3c = b3[:, None]
    acc = _stage1(title_t, num_t, w1t_t, w1n_t, b1c)
    return _stage2(acc, dom_t, usr_t, w1d_t, w1u_t, w2_t, b2c,
                   w3_t, b3c)


# speedup vs baseline: 5.4281x; 1.0137x over previous
import jax
import jax.numpy as jnp
from jax.experimental import pallas as pl
from jax.experimental.pallas import tpu as pltpu

BATCH = 16384
TITLE_DIM = 200
NUM_DIM = 36
DOMAIN_DIM = 16
USER_DIM = 24
HIDDEN = 128


def _full(a):
    return pl.BlockSpec(a.shape, lambda i: (0,) * a.ndim)


def _stage1_body(title_ref, num_ref, w1t_ref, w1n_ref, b1_ref, acc_ref):
    bf = jnp.bfloat16
    f32 = jnp.float32
    acc = jnp.dot(w1t_ref[...].astype(bf), title_ref[...].astype(bf),
                  preferred_element_type=f32)
    acc += jnp.dot(w1n_ref[...].astype(bf), num_ref[...].astype(bf),
                   preferred_element_type=f32)
    acc_ref[...] = (acc + b1_ref[...]).astype(bf)


def _stage1(title_t, num_t, w1t_t, w1n_t, b1c, block_m=8192):
    grid = (BATCH // block_m,)
    return pl.pallas_call(
        _stage1_body,
        grid=grid,
        in_specs=[
            pl.BlockSpec((TITLE_DIM, block_m), lambda i: (0, i)),
            pl.BlockSpec((NUM_DIM, block_m), lambda i: (0, i)),
            _full(w1t_t), _full(w1n_t), _full(b1c),
        ],
        out_specs=pl.BlockSpec((HIDDEN, block_m), lambda i: (0, i)),
        out_shape=jax.ShapeDtypeStruct((HIDDEN, BATCH), jnp.bfloat16),
        compiler_params=pltpu.CompilerParams(
            dimension_semantics=("parallel",)),
    )(title_t, num_t, w1t_t, w1n_t, b1c)


def _stage2_body(acc_ref, dom_ref, usr_ref, w1d_ref, w1u_ref,
                 w2_ref, b2_ref, w3_ref, b3_ref, out_ref):
    bf = jnp.bfloat16
    f32 = jnp.float32
    acc = acc_ref[...].astype(f32)
    acc += jnp.dot(w1d_ref[...].astype(bf), dom_ref[...].astype(bf),
                   preferred_element_type=f32)
    acc += jnp.dot(w1u_ref[...].astype(bf), usr_ref[...].astype(bf),
                   preferred_element_type=f32)
    h1 = jnp.maximum(acc, 0.0)
    h2 = jnp.maximum(
        jnp.dot(w2_ref[...].astype(bf), h1.astype(bf),
                preferred_element_type=f32) + b2_ref[...], 0.0)
    out = jnp.dot(w3_ref[...].astype(bf), h2.astype(bf),
                  preferred_element_type=f32) + b3_ref[...]
    out_ref[...] = out[0]


def _stage2(acc, dom_t, usr_t, w1d_t, w1u_t, w2_t, b2c, w3_t, b3c,
            block_m=8192):
    grid = (BATCH // block_m,)
    return pl.pallas_call(
        _stage2_body,
        grid=grid,
        in_specs=[
            pl.BlockSpec((HIDDEN, block_m), lambda i: (0, i)),
            pl.BlockSpec((DOMAIN_DIM, block_m), lambda i: (0, i)),
            pl.BlockSpec((USER_DIM, block_m), lambda i: (0, i)),
            _full(w1d_t), _full(w1u_t), _full(w2_t), _full(b2c),
            _full(w3_t), _full(b3c),
        ],
        out_specs=pl.BlockSpec((block_m,), lambda i: (i,)),
        out_shape=jax.ShapeDtypeStruct((BATCH,), jnp.float32),
        compiler_params=pltpu.CompilerParams(
            dimension_semantics=("parallel",)),
    )(acc, dom_t, usr_t, w1d_t, w1u_t, w2_t, b2c, w3_t, b3c)


def kernel(title_emb, numerical_features, domain_ids, user_ids,
           domain_table, user_table, W1, b1, W2, b2, W3, b3):
    usr_t = user_table.T.at[:, user_ids].get(mode="promise_in_bounds")
    dom_t = domain_table.T.at[:, domain_ids].get(mode="promise_in_bounds")
    title_t = title_emb.T
    num_t = numerical_features.T
    w1t_t = W1[:TITLE_DIM].T * 0.5
    w1n_t = W1[TITLE_DIM:TITLE_DIM + NUM_DIM].T
    w1d_t = W1[TITLE_DIM + NUM_DIM:TITLE_DIM + NUM_DIM + DOMAIN_DIM].T
    w1u_t = W1[TITLE_DIM + NUM_DIM + DOMAIN_DIM:].T
    w2_t = W2.T
    w3_t = W3.T
    b1c = b1[:, None]
    b2c = b2[:, None]
    b3c = b3[:, None]
    acc = _stage1(title_t, num_t, w1t_t, w1n_t, b1c)
    return _stage2(acc, dom_t, usr_t, w1d_t, w1u_t, w2_t, b2c,
                   w3_t, b3c)
